# Initial kernel scaffold; baseline (speedup 1.0000x reference)
import jax
import jax.numpy as jnp
from jax.experimental import pallas as pl


def kernel(users, pos_items, neg_items, user_embeds, item_embeds, beta_uD, beta_iD, ii_neighbor_mat, ii_constraint_mat):
    raise NotImplementedError("write your pallas kernel here")


# speedup vs baseline: 3.8540x; 3.8540x over previous
import functools

import jax
import jax.numpy as jnp
from jax import lax
from jax.experimental import pallas as pl
from jax.experimental.pallas import tpu as pltpu
from jax.experimental.pallas import tpu_sc as plsc

UN = 100000
IN = 100000
D = 64
BT = 16384
NN = 20
KN = 10
W1 = 1.0
W2 = 1.0
W3 = 1.0
W4 = 1.0
NEG_W = 1.0
GAMMA = 1e-05
LAMB = 1e-05

NWORK = 32
PERW = BT // NWORK
CB = 32
NCH = PERW // CB
NROW16 = UN // 16
KROW16 = UN * KN // 16


def _sc_kernel_fn():
    mesh = plsc.VectorSubcoreMesh(core_axis_name="c", subcore_axis_name="s")
    out_type = (
        jax.ShapeDtypeStruct((BT,), jnp.float32),
        jax.ShapeDtypeStruct((BT,), jnp.float32),
        jax.ShapeDtypeStruct((BT * NN,), jnp.float32),
        jax.ShapeDtypeStruct((BT * NN,), jnp.float32),
        jax.ShapeDtypeStruct((BT * KN,), jnp.float32),
        jax.ShapeDtypeStruct((BT * KN,), jnp.float32),
    )
    scratch = [
        pltpu.VMEM((CB,), jnp.int32),
        pltpu.VMEM((CB,), jnp.int32),
        pltpu.VMEM((CB * NN // 128, 128), jnp.int32),
        pltpu.VMEM((CB,), jnp.int32),
        pltpu.VMEM((CB,), jnp.int32),
        pltpu.VMEM((2 * CB,), jnp.int32),
        pltpu.VMEM((CB * NN // 128, 128), jnp.int32),
        pltpu.VMEM((CB, D), jnp.float32),
        pltpu.VMEM((CB, D), jnp.float32),
        pltpu.VMEM((CB * NN, D), jnp.float32),
        pltpu.VMEM((2 * CB, 16), jnp.int32),
        pltpu.VMEM((2 * CB, 16), jnp.float32),
        pltpu.VMEM((CB, 16), jnp.float32),
        pltpu.VMEM((CB, 16), jnp.float32),
        pltpu.VMEM((CB * NN, 16), jnp.float32),
        pltpu.VMEM((2, 128), jnp.int32),
        pltpu.VMEM((CB * KN - 256,), jnp.int32),
        pltpu.VMEM((CB * KN, D), jnp.float32),
        pltpu.VMEM((CB,), jnp.float32),
        pltpu.VMEM((CB,), jnp.float32),
        pltpu.VMEM((CB * NN,), jnp.float32),
        pltpu.VMEM((CB * NN,), jnp.float32),
        pltpu.VMEM((CB * KN,), jnp.float32),
        pltpu.VMEM((CB * KN,), jnp.float32),
        pltpu.SemaphoreType.DMA,
        pltpu.SemaphoreType.DMA,
    ]

    @functools.partial(pl.kernel, mesh=mesh, out_type=out_type,
                       scratch_types=scratch,
                       compiler_params=pltpu.CompilerParams(
                           needs_layout_passes=False,
                           use_tc_tiling_on_sc=False))
    def body(users_h, pos_h, neg1d_h, uemb_h, iemb_h, bu16_h, bi16_h,
             nbm16_h, cm16_h,
             ps_h, pw_h, ns_h, nw_h, ks_h, sm_h,
             uix_v, pix_v, nix_v, buidx_v, biidx_v, nbmidx_v, binidx_v,
             ue_v, pe_v, ne_v, nbmrows_v, cmrows_v, burows_v, biprows_v,
             binrows_v, nixk_a, nixk_b, nr_v,
             psv, pwv, nsv, nwv, ksv, smv, semA, semB):
        wid = lax.axis_index("s") * 2 + lax.axis_index("c")
        iota = lax.iota(jnp.int32, 16)
        z16i = jnp.zeros((16,), jnp.int32)
        z16f = jnp.zeros((16,), jnp.float32)
        one16 = jnp.ones((16,), jnp.int32)
        c4 = jnp.full((16,), 4, jnp.int32)
        c15 = jnp.full((16,), 15, jnp.int32)
        c7 = jnp.full((16,), 7, jnp.int32)
        c127 = jnp.full((16,), 127, jnp.int32)
        tenv = jnp.full((16,), KN, jnp.int32)
        maxrow = jnp.full((16,), KROW16 - 1, jnp.int32)

        def chunk(it, carry):
            cb = pl.multiple_of(wid * PERW + it * CB, 8)
            pltpu.sync_copy(users_h.at[pl.ds(cb, CB)], uix_v)
            pltpu.sync_copy(pos_h.at[pl.ds(cb, CB)], pix_v)
            nbase = pl.multiple_of(cb * NN, 8)
            for j in range(CB * NN // 128):
                pltpu.sync_copy(neg1d_h.at[pl.ds(nbase + j * 128, 128)],
                                nix_v.at[j])

            for o in range(CB // 16):
                u16 = uix_v[pl.ds(o * 16, 16)]
                buidx_v[pl.ds(o * 16, 16)] = lax.shift_right_logical(u16, c4)
                p16 = pix_v[pl.ds(o * 16, 16)]
                biidx_v[pl.ds(o * 16, 16)] = lax.shift_right_logical(p16, c4)
                w0 = p16 * tenv
                r0 = lax.shift_right_logical(w0, c4)
                nbmidx_v[pl.ds(o * 16, 16)] = r0
                nbmidx_v[pl.ds(CB + o * 16, 16)] = jnp.minimum(r0 + one16,
                                                               maxrow)
            for j in range(CB * NN // 128):
                for o in range(8):
                    n16 = nix_v[j, pl.ds(o * 16, 16)]
                    binidx_v[j, pl.ds(o * 16, 16)] = \
                        lax.shift_right_logical(n16, c4)

            ds = [
                pltpu.async_copy(uemb_h.at[uix_v], ue_v, semA),
                pltpu.async_copy(iemb_h.at[pix_v], pe_v, semA),
                pltpu.async_copy(nbm16_h.at[nbmidx_v], nbmrows_v, semA),
                pltpu.async_copy(cm16_h.at[nbmidx_v], cmrows_v, semA),
                pltpu.async_copy(bu16_h.at[buidx_v], burows_v, semA),
                pltpu.async_copy(bi16_h.at[biidx_v], biprows_v, semA),
            ]
            for j in range(CB * NN // 128):
                ds.append(pltpu.async_copy(
                    iemb_h.at[nix_v.at[j]],
                    ne_v.at[pl.ds(j * 128, 128)], semA))
                ds.append(pltpu.async_copy(
                    bi16_h.at[binidx_v.at[j]],
                    binrows_v.at[pl.ds(j * 128, 128)], semA))
            for d_ in ds:
                d_.wait()

            for t in range(CB * KN // 16):
                flat = iota + jnp.full((16,), t * 16, jnp.int32)
                bb = lax.div(flat, tenv)
                kk = flat - bb * tenv
                pb = plsc.load_gather(pix_v, [bb])
                w0 = pb * tenv
                r0 = lax.shift_right_logical(w0, c4)
                w = w0 + kk
                wr = lax.shift_right_logical(w, c4)
                rowloc = bb + (wr - r0) * jnp.full((16,), CB, jnp.int32)
                col = w & c15
                v = plsc.load_gather(nbmrows_v, [rowloc, col])
                if t < 16:
                    nixk_a[t // 8, pl.ds((t % 8) * 16, 16)] = v
                else:
                    nixk_b[pl.ds((t - 16) * 16, 16)] = v
                smv[pl.ds(t * 16, 16)] = plsc.load_gather(cmrows_v,
                                                          [rowloc, col])

            ds2 = [
                pltpu.async_copy(iemb_h.at[nixk_a.at[0]],
                                 nr_v.at[pl.ds(0, 128)], semB),
                pltpu.async_copy(iemb_h.at[nixk_a.at[1]],
                                 nr_v.at[pl.ds(128, 128)], semB),
                pltpu.async_copy(iemb_h.at[nixk_b],
                                 nr_v.at[pl.ds(256, CB * KN - 256)], semB),
            ]
            for d_ in ds2:
                d_.wait()

            w1v = jnp.full((16,), W1, jnp.float32)
            w2v = jnp.full((16,), W2, jnp.float32)
            w3v = jnp.full((16,), W3, jnp.float32)
            w4v = jnp.full((16,), W4, jnp.float32)
            nnv = jnp.full((16,), NN, jnp.int32)
            for g in range(CB // 16):
                bv = iota + jnp.full((16,), g * 16, jnp.int32)
                u16 = uix_v[pl.ds(g * 16, 16)]
                p16 = pix_v[pl.ds(g * 16, 16)]
                ub = plsc.load_gather(burows_v, [bv, u16 & c15])
                ib = plsc.load_gather(biprows_v, [bv, p16 & c15])

                def pbody(d, c):
                    dsp, acc = c
                    acc = acc + (plsc.load_gather(ue_v, [bv, dsp]) *
                                 plsc.load_gather(pe_v, [bv, dsp]))
                    return (dsp + one16, acc)
                _, accp = lax.fori_loop(0, D, pbody, (z16i, z16f))
                psv[pl.ds(g * 16, 16)] = accp
                pwv[pl.ds(g * 16, 16)] = w1v + w2v * ub * ib

                rows = [bv * nnv + jnp.full((16,), j, jnp.int32)
                        for j in range(NN)]

                def nbody(d, c):
                    dsp, accs = c
                    ud = plsc.load_gather(ue_v, [bv, dsp])
                    accs = tuple(
                        accs[j] + ud * plsc.load_gather(ne_v, [rows[j], dsp])
                        for j in range(NN))
                    return (dsp + one16, accs)
                _, accn = lax.fori_loop(0, D, nbody, (z16i, (z16f,) * NN))
                for j in range(NN):
                    plsc.store_scatter(nsv, [rows[j]], accn[j])
                    nidx = plsc.load_gather(
                        nix_v, [lax.shift_right_logical(rows[j], c7),
                                rows[j] & c127])
                    nb = plsc.load_gather(binrows_v, [rows[j], nidx & c15])
                    plsc.store_scatter(nwv, [rows[j]], w3v + w4v * ub * nb)

                rk = [bv * tenv + jnp.full((16,), k, jnp.int32)
                      for k in range(KN)]

                def kbody(d, c):
                    dsp, accs = c
                    ud = plsc.load_gather(ue_v, [bv, dsp])
                    accs = tuple(
                        accs[k] + ud * plsc.load_gather(nr_v, [rk[k], dsp])
                        for k in range(KN))
                    return (dsp + one16, accs)
                _, acck = lax.fori_loop(0, D, kbody, (z16i, (z16f,) * KN))
                for k in range(KN):
                    plsc.store_scatter(ksv, [rk[k]], acck[k])

            pltpu.sync_copy(psv, ps_h.at[pl.ds(cb, CB)])
            pltpu.sync_copy(pwv, pw_h.at[pl.ds(cb, CB)])
            pltpu.sync_copy(nsv, ns_h.at[pl.ds(nbase, CB * NN)])
            pltpu.sync_copy(nwv, nw_h.at[pl.ds(nbase, CB * NN)])
            kfl = pl.multiple_of(cb * KN, 8)
            pltpu.sync_copy(ksv, ks_h.at[pl.ds(kfl, CB * KN)])
            pltpu.sync_copy(smv, sm_h.at[pl.ds(kfl, CB * KN)])
            return carry

        lax.fori_loop(0, NCH, chunk, 0)

    return body


_sc_kernel = _sc_kernel_fn()


def _softplus(x):
    return jnp.maximum(x, 0.0) + jnp.log1p(jnp.exp(-jnp.abs(x)))


_UBLK = 1000
_NGRID = (UN * D // 128) // _UBLK


def _tc_body(ps, pw, ns, nw, ks, sim, ub, ib, out):
    i = pl.program_id(0)

    @pl.when(i == 0)
    def _():
        t1 = jnp.sum(pw[...] * _softplus(-ps[...]))
        t2 = (NEG_W / NN) * jnp.sum(nw[...] * _softplus(ns[...]))
        t3 = LAMB * jnp.sum(sim[...] * _softplus(-ks[...]))
        out[...] = (t1 + t2 + t3).reshape(1, 1)

    out[...] += ((GAMMA * 0.5) * (jnp.sum(ub[...] * ub[...]) +
                                  jnp.sum(ib[...] * ib[...]))).reshape(1, 1)


def _tc_loss(ps, pw, ns, nw, ks, sim, uemb, iemb):
    full = lambda shp: pl.BlockSpec(shp, lambda i: (0, 0))
    return pl.pallas_call(
        _tc_body,
        grid=(_NGRID,),
        in_specs=[
            full((BT // 128, 128)),
            full((BT // 128, 128)),
            full((BT * NN // 128, 128)),
            full((BT * NN // 128, 128)),
            full((BT * KN // 128, 128)),
            full((BT * KN // 128, 128)),
            pl.BlockSpec((_UBLK, 128), lambda i: (i, 0)),
            pl.BlockSpec((_UBLK, 128), lambda i: (i, 0)),
        ],
        out_specs=pl.BlockSpec((1, 1), lambda i: (0, 0)),
        out_shape=jax.ShapeDtypeStruct((1, 1), jnp.float32),
    )(ps, pw, ns, nw, ks, sim, uemb, iemb)


def kernel(users, pos_items, neg_items, user_embeds, item_embeds,
           beta_uD, beta_iD, ii_neighbor_mat, ii_constraint_mat):
    users = users.astype(jnp.int32)
    pos_items = pos_items.astype(jnp.int32)
    neg1d = neg_items.astype(jnp.int32).reshape(BT * NN)
    bu16 = beta_uD.reshape(NROW16, 16)
    bi16 = beta_iD.reshape(NROW16, 16)
    nbm16 = ii_neighbor_mat.astype(jnp.int32).reshape(KROW16, 16)
    cm16 = ii_constraint_mat.reshape(KROW16, 16)

    ps, pw, nsc, nw, ksc, sim = _sc_kernel(
        users, pos_items, neg1d, user_embeds, item_embeds, bu16, bi16,
        nbm16, cm16)

    loss = _tc_loss(
        ps.reshape(BT // 128, 128),
        pw.reshape(BT // 128, 128),
        nsc.reshape(BT * NN // 128, 128),
        nw.reshape(BT * NN // 128, 128),
        ksc.reshape(BT * KN // 128, 128),
        sim.reshape(BT * KN // 128, 128),
        user_embeds.reshape(UN * D // 128, 128),
        item_embeds.reshape(IN * D // 128, 128),
    )
    return loss[0, 0]


# speedup vs baseline: 3.9349x; 1.0210x over previous
import functools

import jax
import jax.numpy as jnp
from jax import lax
from jax.experimental import pallas as pl
from jax.experimental.pallas import tpu as pltpu
from jax.experimental.pallas import tpu_sc as plsc

UN = 100000
IN = 100000
D = 64
BT = 16384
NN = 20
KN = 10
W1 = 1.0
W2 = 1.0
W3 = 1.0
W4 = 1.0
NEG_W = 1.0
GAMMA = 1e-05
LAMB = 1e-05

NWORK = 32
PERW = BT // NWORK
CB = 32
NCH = PERW // CB
NROW16 = UN // 16
KROW16 = UN * KN // 16


def _sc_kernel_fn():
    mesh = plsc.VectorSubcoreMesh(core_axis_name="c", subcore_axis_name="s")
    out_type = (
        jax.ShapeDtypeStruct((BT,), jnp.float32),
        jax.ShapeDtypeStruct((BT,), jnp.float32),
        jax.ShapeDtypeStruct((BT * NN,), jnp.float32),
        jax.ShapeDtypeStruct((BT * NN,), jnp.float32),
        jax.ShapeDtypeStruct((BT * KN,), jnp.float32),
        jax.ShapeDtypeStruct((BT * KN,), jnp.float32),
    )
    scratch = [
        pltpu.VMEM((CB,), jnp.int32),
        pltpu.VMEM((CB,), jnp.int32),
        pltpu.VMEM((CB * NN // 128, 128), jnp.int32),
        pltpu.VMEM((CB,), jnp.int32),
        pltpu.VMEM((CB,), jnp.int32),
        pltpu.VMEM((2 * CB,), jnp.int32),
        pltpu.VMEM((CB * NN // 128, 128), jnp.int32),
        pltpu.VMEM((CB, D), jnp.float32),
        pltpu.VMEM((CB, D), jnp.float32),
        pltpu.VMEM((CB * NN, D), jnp.float32),
        pltpu.VMEM((2 * CB, 16), jnp.int32),
        pltpu.VMEM((2 * CB, 16), jnp.float32),
        pltpu.VMEM((CB, 16), jnp.float32),
        pltpu.VMEM((CB, 16), jnp.float32),
        pltpu.VMEM((CB * NN, 16), jnp.float32),
        pltpu.VMEM((2, 128), jnp.int32),
        pltpu.VMEM((CB * KN - 256,), jnp.int32),
        pltpu.VMEM((CB * KN, D), jnp.float32),
        pltpu.VMEM((CB,), jnp.float32),
        pltpu.VMEM((CB,), jnp.float32),
        pltpu.VMEM((CB * NN,), jnp.float32),
        pltpu.VMEM((CB * NN,), jnp.float32),
        pltpu.VMEM((CB * KN,), jnp.float32),
        pltpu.VMEM((CB * KN,), jnp.float32),
        pltpu.SemaphoreType.DMA,
        pltpu.SemaphoreType.DMA,
    ]

    @functools.partial(pl.kernel, mesh=mesh, out_type=out_type,
                       scratch_types=scratch,
                       compiler_params=pltpu.CompilerParams(
                           needs_layout_passes=False,
                           use_tc_tiling_on_sc=False))
    def body(users_h, pos_h, neg1d_h, uemb_h, iemb_h, bu16_h, bi16_h,
             nbm16_h, cm16_h,
             ps_h, pw_h, ns_h, nw_h, ks_h, sm_h,
             uix_v, pix_v, nix_v, buidx_v, biidx_v, nbmidx_v, binidx_v,
             ue_v, pe_v, ne_v, nbmrows_v, cmrows_v, burows_v, biprows_v,
             binrows_v, nixk_a, nixk_b, nr_v,
             psv, pwv, nsv, nwv, ksv, smv, semA, semB):
        wid = lax.axis_index("s") * 2 + lax.axis_index("c")
        iota = lax.iota(jnp.int32, 16)
        z16i = jnp.zeros((16,), jnp.int32)
        z16f = jnp.zeros((16,), jnp.float32)
        one16 = jnp.ones((16,), jnp.int32)
        c4 = jnp.full((16,), 4, jnp.int32)
        c15 = jnp.full((16,), 15, jnp.int32)
        c7 = jnp.full((16,), 7, jnp.int32)
        c127 = jnp.full((16,), 127, jnp.int32)
        tenv = jnp.full((16,), KN, jnp.int32)
        maxrow = jnp.full((16,), KROW16 - 1, jnp.int32)

        def chunk(it, carry):
            cb = pl.multiple_of(wid * PERW + it * CB, 8)
            pltpu.sync_copy(users_h.at[pl.ds(cb, CB)], uix_v)
            pltpu.sync_copy(pos_h.at[pl.ds(cb, CB)], pix_v)
            nbase = pl.multiple_of(cb * NN, 8)
            for j in range(CB * NN // 128):
                pltpu.sync_copy(neg1d_h.at[pl.ds(nbase + j * 128, 128)],
                                nix_v.at[j])

            for o in range(CB // 16):
                u16 = uix_v[pl.ds(o * 16, 16)]
                buidx_v[pl.ds(o * 16, 16)] = lax.shift_right_logical(u16, c4)
                p16 = pix_v[pl.ds(o * 16, 16)]
                biidx_v[pl.ds(o * 16, 16)] = lax.shift_right_logical(p16, c4)
                w0 = p16 * tenv
                r0 = lax.shift_right_logical(w0, c4)
                nbmidx_v[pl.ds(o * 16, 16)] = r0
                nbmidx_v[pl.ds(CB + o * 16, 16)] = jnp.minimum(r0 + one16,
                                                               maxrow)
            for j in range(CB * NN // 128):
                for o in range(8):
                    n16 = nix_v[j, pl.ds(o * 16, 16)]
                    binidx_v[j, pl.ds(o * 16, 16)] = \
                        lax.shift_right_logical(n16, c4)

            ds = [
                pltpu.async_copy(uemb_h.at[uix_v], ue_v, semA),
                pltpu.async_copy(iemb_h.at[pix_v], pe_v, semA),
                pltpu.async_copy(nbm16_h.at[nbmidx_v], nbmrows_v, semA),
                pltpu.async_copy(cm16_h.at[nbmidx_v], cmrows_v, semA),
                pltpu.async_copy(bu16_h.at[buidx_v], burows_v, semA),
                pltpu.async_copy(bi16_h.at[biidx_v], biprows_v, semA),
            ]
            for j in range(CB * NN // 128):
                ds.append(pltpu.async_copy(
                    iemb_h.at[nix_v.at[j]],
                    ne_v.at[pl.ds(j * 128, 128)], semA))
                ds.append(pltpu.async_copy(
                    bi16_h.at[binidx_v.at[j]],
                    binrows_v.at[pl.ds(j * 128, 128)], semA))
            for d_ in ds:
                d_.wait()

            for t in range(CB * KN // 16):
                flat = iota + jnp.full((16,), t * 16, jnp.int32)
                bb = lax.div(flat, tenv)
                kk = flat - bb * tenv
                pb = plsc.load_gather(pix_v, [bb])
                w0 = pb * tenv
                r0 = lax.shift_right_logical(w0, c4)
                w = w0 + kk
                wr = lax.shift_right_logical(w, c4)
                rowloc = bb + (wr - r0) * jnp.full((16,), CB, jnp.int32)
                col = w & c15
                v = plsc.load_gather(nbmrows_v, [rowloc, col])
                if t < 16:
                    nixk_a[t // 8, pl.ds((t % 8) * 16, 16)] = v
                else:
                    nixk_b[pl.ds((t - 16) * 16, 16)] = v
                smv[pl.ds(t * 16, 16)] = plsc.load_gather(cmrows_v,
                                                          [rowloc, col])

            ds2 = [
                pltpu.async_copy(iemb_h.at[nixk_a.at[0]],
                                 nr_v.at[pl.ds(0, 128)], semB),
                pltpu.async_copy(iemb_h.at[nixk_a.at[1]],
                                 nr_v.at[pl.ds(128, 128)], semB),
                pltpu.async_copy(iemb_h.at[nixk_b],
                                 nr_v.at[pl.ds(256, CB * KN - 256)], semB),
            ]

            w1v = jnp.full((16,), W1, jnp.float32)
            w2v = jnp.full((16,), W2, jnp.float32)
            w3v = jnp.full((16,), W3, jnp.float32)
            w4v = jnp.full((16,), W4, jnp.float32)
            nnv = jnp.full((16,), NN, jnp.int32)
            for g in range(CB // 16):
                bv = iota + jnp.full((16,), g * 16, jnp.int32)
                u16 = uix_v[pl.ds(g * 16, 16)]
                p16 = pix_v[pl.ds(g * 16, 16)]
                ub = plsc.load_gather(burows_v, [bv, u16 & c15])
                ib = plsc.load_gather(biprows_v, [bv, p16 & c15])

                def pbody(d, c):
                    dsp, acc = c
                    acc = acc + (plsc.load_gather(ue_v, [bv, dsp]) *
                                 plsc.load_gather(pe_v, [bv, dsp]))
                    return (dsp + one16, acc)
                _, accp = lax.fori_loop(0, D, pbody, (z16i, z16f))
                psv[pl.ds(g * 16, 16)] = accp
                pwv[pl.ds(g * 16, 16)] = w1v + w2v * ub * ib

                rows = [bv * nnv + jnp.full((16,), j, jnp.int32)
                        for j in range(NN)]

                def nbody(d, c):
                    dsp, accs = c
                    ud = plsc.load_gather(ue_v, [bv, dsp])
                    accs = tuple(
                        accs[j] + ud * plsc.load_gather(ne_v, [rows[j], dsp])
                        for j in range(NN))
                    return (dsp + one16, accs)
                _, accn = lax.fori_loop(0, D, nbody, (z16i, (z16f,) * NN))
                for j in range(NN):
                    plsc.store_scatter(nsv, [rows[j]], accn[j])
                    nidx = plsc.load_gather(
                        nix_v, [lax.shift_right_logical(rows[j], c7),
                                rows[j] & c127])
                    nb = plsc.load_gather(binrows_v, [rows[j], nidx & c15])
                    plsc.store_scatter(nwv, [rows[j]], w3v + w4v * ub * nb)

                if g == 0:
                    for d_ in ds2:
                        d_.wait()
                rk = [bv * tenv + jnp.full((16,), k, jnp.int32)
                      for k in range(KN)]

                def kbody(d, c):
                    dsp, accs = c
                    ud = plsc.load_gather(ue_v, [bv, dsp])
                    accs = tuple(
                        accs[k] + ud * plsc.load_gather(nr_v, [rk[k], dsp])
                        for k in range(KN))
                    return (dsp + one16, accs)
                _, acck = lax.fori_loop(0, D, kbody, (z16i, (z16f,) * KN))
                for k in range(KN):
                    plsc.store_scatter(ksv, [rk[k]], acck[k])

            pltpu.sync_copy(psv, ps_h.at[pl.ds(cb, CB)])
            pltpu.sync_copy(pwv, pw_h.at[pl.ds(cb, CB)])
            pltpu.sync_copy(nsv, ns_h.at[pl.ds(nbase, CB * NN)])
            pltpu.sync_copy(nwv, nw_h.at[pl.ds(nbase, CB * NN)])
            kfl = pl.multiple_of(cb * KN, 8)
            pltpu.sync_copy(ksv, ks_h.at[pl.ds(kfl, CB * KN)])
            pltpu.sync_copy(smv, sm_h.at[pl.ds(kfl, CB * KN)])
            return carry

        lax.fori_loop(0, NCH, chunk, 0)

    return body


_sc_kernel = _sc_kernel_fn()


def _softplus(x):
    return jnp.maximum(x, 0.0) + jnp.log1p(jnp.exp(-jnp.abs(x)))


_UBLK = 1000
_NGRID = (UN * D // 128) // _UBLK


def _tc_body(ps, pw, ns, nw, ks, sim, ub, ib, out):
    i = pl.program_id(0)

    @pl.when(i == 0)
    def _():
        t1 = jnp.sum(pw[...] * _softplus(-ps[...]))
        t2 = (NEG_W / NN) * jnp.sum(nw[...] * _softplus(ns[...]))
        t3 = LAMB * jnp.sum(sim[...] * _softplus(-ks[...]))
        out[...] = (t1 + t2 + t3).reshape(1, 1)

    out[...] += ((GAMMA * 0.5) * (jnp.sum(ub[...] * ub[...]) +
                                  jnp.sum(ib[...] * ib[...]))).reshape(1, 1)


def _tc_loss(ps, pw, ns, nw, ks, sim, uemb, iemb):
    full = lambda shp: pl.BlockSpec(shp, lambda i: (0, 0))
    return pl.pallas_call(
        _tc_body,
        grid=(_NGRID,),
        in_specs=[
            full((BT // 128, 128)),
            full((BT // 128, 128)),
            full((BT * NN // 128, 128)),
            full((BT * NN // 128, 128)),
            full((BT * KN // 128, 128)),
            full((BT * KN // 128, 128)),
            pl.BlockSpec((_UBLK, 128), lambda i: (i, 0)),
            pl.BlockSpec((_UBLK, 128), lambda i: (i, 0)),
        ],
        out_specs=pl.BlockSpec((1, 1), lambda i: (0, 0)),
        out_shape=jax.ShapeDtypeStruct((1, 1), jnp.float32),
    )(ps, pw, ns, nw, ks, sim, uemb, iemb)


def kernel(users, pos_items, neg_items, user_embeds, item_embeds,
           beta_uD, beta_iD, ii_neighbor_mat, ii_constraint_mat):
    users = users.astype(jnp.int32)
    pos_items = pos_items.astype(jnp.int32)
    neg1d = neg_items.astype(jnp.int32).reshape(BT * NN)
    bu16 = beta_uD.reshape(NROW16, 16)
    bi16 = beta_iD.reshape(NROW16, 16)
    nbm16 = ii_neighbor_mat.astype(jnp.int32).reshape(KROW16, 16)
    cm16 = ii_constraint_mat.reshape(KROW16, 16)

    ps, pw, nsc, nw, ksc, sim = _sc_kernel(
        users, pos_items, neg1d, user_embeds, item_embeds, bu16, bi16,
        nbm16, cm16)

    loss = _tc_loss(
        ps.reshape(BT // 128, 128),
        pw.reshape(BT // 128, 128),
        nsc.reshape(BT * NN // 128, 128),
        nw.reshape(BT * NN // 128, 128),
        ksc.reshape(BT * KN // 128, 128),
        sim.reshape(BT * KN // 128, 128),
        user_embeds.reshape(UN * D // 128, 128),
        item_embeds.reshape(IN * D // 128, 128),
    )
    return loss[0, 0]


# speedup vs baseline: 4.1038x; 1.0429x over previous
import functools

import jax
import jax.numpy as jnp
from jax import lax
from jax.experimental import pallas as pl
from jax.experimental.pallas import tpu as pltpu
from jax.experimental.pallas import tpu_sc as plsc

UN = 100000
IN = 100000
D = 64
BT = 16384
NN = 20
KN = 10
W1 = 1.0
W2 = 1.0
W3 = 1.0
W4 = 1.0
NEG_W = 1.0
GAMMA = 1e-05
LAMB = 1e-05

NWORK = 32
PERW = BT // NWORK
CB = 32
NCH = PERW // CB
NROW16 = UN // 16
KROW16 = UN * KN // 16


def _sc_kernel_fn():
    mesh = plsc.VectorSubcoreMesh(core_axis_name="c", subcore_axis_name="s")
    out_type = (
        jax.ShapeDtypeStruct((BT,), jnp.float32),
        jax.ShapeDtypeStruct((BT,), jnp.float32),
        jax.ShapeDtypeStruct((BT * NN,), jnp.float32),
        jax.ShapeDtypeStruct((BT * NN,), jnp.float32),
        jax.ShapeDtypeStruct((BT * KN,), jnp.float32),
        jax.ShapeDtypeStruct((BT * KN,), jnp.float32),
    )
    scratch = [
        pltpu.VMEM((CB,), jnp.int32),
        pltpu.VMEM((CB,), jnp.int32),
        pltpu.VMEM((CB * NN // 128, 128), jnp.int32),
        pltpu.VMEM((CB,), jnp.int32),
        pltpu.VMEM((CB,), jnp.int32),
        pltpu.VMEM((2 * CB,), jnp.int32),
        pltpu.VMEM((CB * NN // 128, 128), jnp.int32),
        pltpu.VMEM((CB, D), jnp.float32),
        pltpu.VMEM((CB, D), jnp.float32),
        pltpu.VMEM((CB * NN, D), jnp.float32),
        pltpu.VMEM((2 * CB, 16), jnp.int32),
        pltpu.VMEM((2 * CB, 16), jnp.float32),
        pltpu.VMEM((CB, 16), jnp.float32),
        pltpu.VMEM((CB, 16), jnp.float32),
        pltpu.VMEM((CB * NN, 16), jnp.float32),
        pltpu.VMEM((2, 128), jnp.int32),
        pltpu.VMEM((CB * KN - 256,), jnp.int32),
        pltpu.VMEM((CB * KN, D), jnp.float32),
        pltpu.VMEM((CB,), jnp.float32),
        pltpu.VMEM((CB,), jnp.float32),
        pltpu.VMEM((CB * NN,), jnp.float32),
        pltpu.VMEM((CB * NN,), jnp.float32),
        pltpu.VMEM((CB * KN,), jnp.float32),
        pltpu.VMEM((CB * KN,), jnp.float32),
        pltpu.SemaphoreType.DMA,
        pltpu.SemaphoreType.DMA,
    ]

    @functools.partial(pl.kernel, mesh=mesh, out_type=out_type,
                       scratch_types=scratch,
                       compiler_params=pltpu.CompilerParams(
                           needs_layout_passes=False,
                           use_tc_tiling_on_sc=False))
    def body(users_h, pos_h, neg1d_h, uemb_h, iemb_h, bu16_h, bi16_h,
             nbm16_h, cm16_h,
             ps_h, pw_h, ns_h, nw_h, ks_h, sm_h,
             uix_v, pix_v, nix_v, buidx_v, biidx_v, nbmidx_v, binidx_v,
             ue_v, pe_v, ne_v, nbmrows_v, cmrows_v, burows_v, biprows_v,
             binrows_v, nixk_a, nixk_b, nr_v,
             psv, pwv, nsv, nwv, ksv, smv, semA, semB):
        wid = lax.axis_index("s") * 2 + lax.axis_index("c")
        iota = lax.iota(jnp.int32, 16)
        z16i = jnp.zeros((16,), jnp.int32)
        z16f = jnp.zeros((16,), jnp.float32)
        one16 = jnp.ones((16,), jnp.int32)
        c4 = jnp.full((16,), 4, jnp.int32)
        c15 = jnp.full((16,), 15, jnp.int32)
        c7 = jnp.full((16,), 7, jnp.int32)
        c127 = jnp.full((16,), 127, jnp.int32)
        tenv = jnp.full((16,), KN, jnp.int32)
        maxrow = jnp.full((16,), KROW16 - 1, jnp.int32)

        def chunk(it, carry):
            cb = pl.multiple_of(wid * PERW + it * CB, 8)
            nbase = pl.multiple_of(cb * NN, 8)
            ds0 = [
                pltpu.async_copy(users_h.at[pl.ds(cb, CB)], uix_v, semB),
                pltpu.async_copy(pos_h.at[pl.ds(cb, CB)], pix_v, semB),
            ]
            for j in range(CB * NN // 128):
                ds0.append(pltpu.async_copy(
                    neg1d_h.at[pl.ds(nbase + j * 128, 128)],
                    nix_v.at[j], semB))
            for d_ in ds0:
                d_.wait()

            for o in range(CB // 16):
                u16 = uix_v[pl.ds(o * 16, 16)]
                buidx_v[pl.ds(o * 16, 16)] = lax.shift_right_logical(u16, c4)
                p16 = pix_v[pl.ds(o * 16, 16)]
                biidx_v[pl.ds(o * 16, 16)] = lax.shift_right_logical(p16, c4)
                w0 = p16 * tenv
                r0 = lax.shift_right_logical(w0, c4)
                nbmidx_v[pl.ds(o * 16, 16)] = r0
                nbmidx_v[pl.ds(CB + o * 16, 16)] = jnp.minimum(r0 + one16,
                                                               maxrow)
            for j in range(CB * NN // 128):
                for o in range(8):
                    n16 = nix_v[j, pl.ds(o * 16, 16)]
                    binidx_v[j, pl.ds(o * 16, 16)] = \
                        lax.shift_right_logical(n16, c4)

            ds = [
                pltpu.async_copy(uemb_h.at[uix_v], ue_v, semA),
                pltpu.async_copy(iemb_h.at[pix_v], pe_v, semA),
                pltpu.async_copy(nbm16_h.at[nbmidx_v], nbmrows_v, semA),
                pltpu.async_copy(cm16_h.at[nbmidx_v], cmrows_v, semA),
                pltpu.async_copy(bu16_h.at[buidx_v], burows_v, semA),
                pltpu.async_copy(bi16_h.at[biidx_v], biprows_v, semA),
            ]
            for j in range(CB * NN // 128):
                ds.append(pltpu.async_copy(
                    iemb_h.at[nix_v.at[j]],
                    ne_v.at[pl.ds(j * 128, 128)], semA))
                ds.append(pltpu.async_copy(
                    bi16_h.at[binidx_v.at[j]],
                    binrows_v.at[pl.ds(j * 128, 128)], semA))
            for d_ in ds:
                d_.wait()

            for t in range(CB * KN // 16):
                flat = iota + jnp.full((16,), t * 16, jnp.int32)
                bb = lax.div(flat, tenv)
                kk = flat - bb * tenv
                pb = plsc.load_gather(pix_v, [bb])
                w0 = pb * tenv
                r0 = lax.shift_right_logical(w0, c4)
                w = w0 + kk
                wr = lax.shift_right_logical(w, c4)
                rowloc = bb + (wr - r0) * jnp.full((16,), CB, jnp.int32)
                col = w & c15
                v = plsc.load_gather(nbmrows_v, [rowloc, col])
                if t < 16:
                    nixk_a[t // 8, pl.ds((t % 8) * 16, 16)] = v
                else:
                    nixk_b[pl.ds((t - 16) * 16, 16)] = v
                smv[pl.ds(t * 16, 16)] = plsc.load_gather(cmrows_v,
                                                          [rowloc, col])

            ds2 = [
                pltpu.async_copy(iemb_h.at[nixk_a.at[0]],
                                 nr_v.at[pl.ds(0, 128)], semB),
                pltpu.async_copy(iemb_h.at[nixk_a.at[1]],
                                 nr_v.at[pl.ds(128, 128)], semB),
                pltpu.async_copy(iemb_h.at[nixk_b],
                                 nr_v.at[pl.ds(256, CB * KN - 256)], semB),
            ]

            w1v = jnp.full((16,), W1, jnp.float32)
            w2v = jnp.full((16,), W2, jnp.float32)
            w3v = jnp.full((16,), W3, jnp.float32)
            w4v = jnp.full((16,), W4, jnp.float32)
            nnv = jnp.full((16,), NN, jnp.int32)
            for g in range(CB // 16):
                bv = iota + jnp.full((16,), g * 16, jnp.int32)
                u16 = uix_v[pl.ds(g * 16, 16)]
                p16 = pix_v[pl.ds(g * 16, 16)]
                ub = plsc.load_gather(burows_v, [bv, u16 & c15])
                ib = plsc.load_gather(biprows_v, [bv, p16 & c15])

                def pbody(d, c):
                    dsp, acc = c
                    acc = acc + (plsc.load_gather(ue_v, [bv, dsp]) *
                                 plsc.load_gather(pe_v, [bv, dsp]))
                    return (dsp + one16, acc)
                _, accp = lax.fori_loop(0, D, pbody, (z16i, z16f))
                psv[pl.ds(g * 16, 16)] = accp
                pwv[pl.ds(g * 16, 16)] = w1v + w2v * ub * ib

                rows = [bv * nnv + jnp.full((16,), j, jnp.int32)
                        for j in range(NN)]

                def nbody(d, c):
                    dsp, accs = c
                    ud = plsc.load_gather(ue_v, [bv, dsp])
                    accs = tuple(
                        accs[j] + ud * plsc.load_gather(ne_v, [rows[j], dsp])
                        for j in range(NN))
                    return (dsp + one16, accs)
                _, accn = lax.fori_loop(0, D, nbody, (z16i, (z16f,) * NN))
                for j in range(NN):
                    plsc.store_scatter(nsv, [rows[j]], accn[j])
                    nidx = plsc.load_gather(
                        nix_v, [lax.shift_right_logical(rows[j], c7),
                                rows[j] & c127])
                    nb = plsc.load_gather(binrows_v, [rows[j], nidx & c15])
                    plsc.store_scatter(nwv, [rows[j]], w3v + w4v * ub * nb)

                if g == 0:
                    for d_ in ds2:
                        d_.wait()
                rk = [bv * tenv + jnp.full((16,), k, jnp.int32)
                      for k in range(KN)]

                def kbody(d, c):
                    dsp, accs = c
                    ud = plsc.load_gather(ue_v, [bv, dsp])
                    accs = tuple(
                        accs[k] + ud * plsc.load_gather(nr_v, [rk[k], dsp])
                        for k in range(KN))
                    return (dsp + one16, accs)
                _, acck = lax.fori_loop(0, D, kbody, (z16i, (z16f,) * KN))
                for k in range(KN):
                    plsc.store_scatter(ksv, [rk[k]], acck[k])

            kfl = pl.multiple_of(cb * KN, 8)
            ds3 = [
                pltpu.async_copy(psv, ps_h.at[pl.ds(cb, CB)], semB),
                pltpu.async_copy(pwv, pw_h.at[pl.ds(cb, CB)], semB),
                pltpu.async_copy(nsv, ns_h.at[pl.ds(nbase, CB * NN)], semB),
                pltpu.async_copy(nwv, nw_h.at[pl.ds(nbase, CB * NN)], semB),
                pltpu.async_copy(ksv, ks_h.at[pl.ds(kfl, CB * KN)], semB),
                pltpu.async_copy(smv, sm_h.at[pl.ds(kfl, CB * KN)], semB),
            ]
            for d_ in ds3:
                d_.wait()
            return carry

        lax.fori_loop(0, NCH, chunk, 0)

    return body


_sc_kernel = _sc_kernel_fn()


def _softplus(x):
    return jnp.maximum(x, 0.0) + jnp.log1p(jnp.exp(-jnp.abs(x)))


_UBLK = 1000
_NGRID = (UN * D // 128) // _UBLK


def _tc_body(ps, pw, ns, nw, ks, sim, ub, ib, out):
    i = pl.program_id(0)

    @pl.when(i == 0)
    def _():
        t1 = jnp.sum(pw[...] * _softplus(-ps[...]))
        t2 = (NEG_W / NN) * jnp.sum(nw[...] * _softplus(ns[...]))
        t3 = LAMB * jnp.sum(sim[...] * _softplus(-ks[...]))
        out[...] = (t1 + t2 + t3).reshape(1, 1)

    out[...] += ((GAMMA * 0.5) * (jnp.sum(ub[...] * ub[...]) +
                                  jnp.sum(ib[...] * ib[...]))).reshape(1, 1)


def _tc_loss(ps, pw, ns, nw, ks, sim, uemb, iemb):
    full = lambda shp: pl.BlockSpec(shp, lambda i: (0, 0))
    return pl.pallas_call(
        _tc_body,
        grid=(_NGRID,),
        in_specs=[
            full((BT // 128, 128)),
            full((BT // 128, 128)),
            full((BT * NN // 128, 128)),
            full((BT * NN // 128, 128)),
            full((BT * KN // 128, 128)),
            full((BT * KN // 128, 128)),
            pl.BlockSpec((_UBLK, 128), lambda i: (i, 0)),
            pl.BlockSpec((_UBLK, 128), lambda i: (i, 0)),
        ],
        out_specs=pl.BlockSpec((1, 1), lambda i: (0, 0)),
        out_shape=jax.ShapeDtypeStruct((1, 1), jnp.float32),
    )(ps, pw, ns, nw, ks, sim, uemb, iemb)


def kernel(users, pos_items, neg_items, user_embeds, item_embeds,
           beta_uD, beta_iD, ii_neighbor_mat, ii_constraint_mat):
    users = users.astype(jnp.int32)
    pos_items = pos_items.astype(jnp.int32)
    neg1d = neg_items.astype(jnp.int32).reshape(BT * NN)
    bu16 = beta_uD.reshape(NROW16, 16)
    bi16 = beta_iD.reshape(NROW16, 16)
    nbm16 = ii_neighbor_mat.astype(jnp.int32).reshape(KROW16, 16)
    cm16 = ii_constraint_mat.reshape(KROW16, 16)

    ps, pw, nsc, nw, ksc, sim = _sc_kernel(
        users, pos_items, neg1d, user_embeds, item_embeds, bu16, bi16,
        nbm16, cm16)

    loss = _tc_loss(
        ps.reshape(BT // 128, 128),
        pw.reshape(BT // 128, 128),
        nsc.reshape(BT * NN // 128, 128),
        nw.reshape(BT * NN // 128, 128),
        ksc.reshape(BT * KN // 128, 128),
        sim.reshape(BT * KN // 128, 128),
        user_embeds.reshape(UN * D // 128, 128),
        item_embeds.reshape(IN * D // 128, 128),
    )
    return loss[0, 0]
